# baseline (speedup 1.0000x reference)
import functools

import jax
import jax.numpy as jnp
from jax import lax
from jax.experimental import pallas as pl
from jax.experimental.pallas import tpu as pltpu
from jax.experimental.pallas import tpu_sc as plsc

_NC = 2
_NS = 16
_NW = _NC * _NS
_NBUF = 4


@functools.lru_cache(maxsize=None)
def _build_gather(b: int, n_neg: int, d: int):
    pos_per_w = b // _NW
    rows_per_w = b // _NW
    assert b % _NW == 0 and pos_per_w % 8 == 0 and n_neg % 8 == 0

    mesh = plsc.VectorSubcoreMesh(
        core_axis_name="c", subcore_axis_name="s",
        num_cores=_NC, num_subcores=_NS)

    @functools.partial(
        pl.kernel,
        out_type=(
            jax.ShapeDtypeStruct((b, d), jnp.float32),
            jax.ShapeDtypeStruct((b, n_neg, d), jnp.float32),
        ),
        mesh=mesh,
        scratch_types=[
            pltpu.VMEM((pos_per_w,), jnp.int32),
            pltpu.VMEM((pos_per_w, d), jnp.float32),
            [pltpu.VMEM((n_neg,), jnp.int32) for _ in range(_NBUF)],
            [pltpu.VMEM((n_neg, d), jnp.float32) for _ in range(_NBUF)],
            [pltpu.SemaphoreType.DMA for _ in range(_NBUF)],
            pltpu.SemaphoreType.DMA,
        ],
        compiler_params=pltpu.CompilerParams(use_tc_tiling_on_sc=False),
    )
    def gather_k(pos_hbm, neg_hbm, table_hbm, pos_out, neg_out,
                 pidx_v, prows_v, idxs, rows, gsems, psem):
        wid = lax.axis_index("s") * _NC + lax.axis_index("c")
        cbase = wid * rows_per_w

        for bi in range(_NBUF):
            pltpu.sync_copy(neg_hbm.at[cbase + bi], idxs[bi])
            pltpu.async_copy(table_hbm.at[idxs[bi]], rows[bi], gsems[bi])

        pbase = wid * pos_per_w
        pltpu.sync_copy(pos_hbm.at[pl.ds(pbase, pos_per_w)], pidx_v)
        pltpu.async_copy(table_hbm.at[pidx_v], prows_v, psem).wait()
        pltpu.sync_copy(prows_v, pos_out.at[pl.ds(pbase, pos_per_w)])

        @pl.loop(0, rows_per_w, step=_NBUF)
        def _ring(j):
            for bi in range(_NBUF):
                cj = cbase + j + bi
                pltpu.make_async_copy(table_hbm.at[idxs[bi]], rows[bi],
                                      gsems[bi]).wait()
                pltpu.sync_copy(rows[bi], neg_out.at[cj])
                nxt = cj + _NBUF

                @pl.when(j + bi + _NBUF < rows_per_w)
                def _refill():
                    pltpu.sync_copy(neg_hbm.at[nxt], idxs[bi])
                    pltpu.async_copy(table_hbm.at[idxs[bi]], rows[bi],
                                     gsems[bi])

    return gather_k


def kernel(query_emb, positive_ids, negative_ids, table):
    b, n_neg = negative_ids.shape
    _, d = table.shape
    gather_k = _build_gather(b, n_neg, d)
    pos_emb, neg_emb = gather_k(positive_ids, negative_ids, table)
    return (query_emb, pos_emb, neg_emb)
